# Initial kernel scaffold; baseline (speedup 1.0000x reference)
import jax
import jax.numpy as jnp
from jax.experimental import pallas as pl


def kernel(x, coordinate, Wq, bq, Wk, bk, Wv, bv, Wo, bo):
    raise NotImplementedError("write your pallas kernel here")


# speedup vs baseline: 10.7059x; 10.7059x over previous
import functools

import jax
import jax.numpy as jnp
import numpy as np
from jax import lax
from jax.experimental import pallas as pl
from jax.experimental.pallas import tpu as pltpu
from jax.experimental.pallas import tpu_sc as plsc

D_MODEL = 512
N_HEADS = 8
HEAD_DIM = D_MODEL // N_HEADS
K_NB = 16
NEG_BIG = 3.0e38
IDX_BIG = 2 ** 30


def _qkv_body(x_ref, wq_ref, bq_ref, wk_ref, bk_ref, wv_ref, bv_ref,
              q_ref, k_ref, v_ref):
    x = x_ref[...]
    dot = lambda a, b: lax.dot_general(
        a, b, (((1,), (0,)), ((), ())), preferred_element_type=jnp.float32)
    q_ref[...] = dot(x, wq_ref[...]) + bq_ref[...]
    k_ref[...] = dot(x, wk_ref[...]) + bk_ref[...]
    v_ref[...] = dot(x, wv_ref[...]) + bv_ref[...]


def _qkv(xf, Wq, bq, Wk, bk, Wv, bv):
    m = xf.shape[0]
    blk = 512
    grid = (m // blk,)
    row_spec = pl.BlockSpec((blk, D_MODEL), lambda i: (i, 0))
    full_spec = pl.BlockSpec((D_MODEL, D_MODEL), lambda i: (0, 0))
    bias_spec = pl.BlockSpec((1, D_MODEL), lambda i: (0, 0))
    out = jax.ShapeDtypeStruct((m, D_MODEL), jnp.float32)
    return pl.pallas_call(
        _qkv_body,
        grid=grid,
        in_specs=[row_spec, full_spec, bias_spec, full_spec, bias_spec,
                  full_spec, bias_spec],
        out_specs=[row_spec, row_spec, row_spec],
        out_shape=[out, out, out],
    )(xf, Wq, bq.reshape(1, -1), Wk, bk.reshape(1, -1), Wv, bv.reshape(1, -1))


def _knn_body(cp_ref, cpt_ref, idx_ref, *, rows, n):
    b = pl.program_id(0)
    cb = cp_ref[0]
    cft = cpt_ref[0]
    g = lax.dot_general(cb, cft, (((1,), (0,)), ((), ())),
                        preferred_element_type=jnp.float32)
    sqb = jnp.sum(cb * cb, axis=1, keepdims=True)
    sqf = jnp.sum(cft * cft, axis=0, keepdims=True)
    d2 = sqb + sqf - 2.0 * g
    iota = lax.broadcasted_iota(jnp.int32, (rows, n), 1)
    cols = []
    for _ in range(K_NB):
        m = jnp.min(d2, axis=1, keepdims=True)
        cand = jnp.where(d2 == m, iota, IDX_BIG)
        amin = jnp.min(cand, axis=1, keepdims=True)
        cols.append(amin)
        d2 = jnp.where(iota == amin, NEG_BIG, d2)
    idx_ref[...] = jnp.concatenate(cols, axis=1) + b * n


def _knn(coordinate):
    B, n, _ = coordinate.shape
    rows = 256
    cpad = jnp.zeros((B, n, 128), jnp.float32).at[:, :, :3].set(coordinate)
    cpadT = cpad.transpose(0, 2, 1)
    grid = (B, n // rows)
    return pl.pallas_call(
        functools.partial(_knn_body, rows=rows, n=n),
        grid=grid,
        in_specs=[
            pl.BlockSpec((1, rows, 128), lambda b, i: (b, i, 0)),
            pl.BlockSpec((1, 128, n), lambda b, i: (b, 0, 0)),
        ],
        out_specs=pl.BlockSpec((rows, K_NB), lambda b, i: (b * (n // rows) + i, 0)),
        out_shape=jax.ShapeDtypeStruct((B * n, K_NB), jnp.int32),
    )(cpad, cpadT)


_NC = 2
_NS = 16
_NW = _NC * _NS


def _gather_sc(kf, vf, gidx):
    total = gidx.shape[0]
    b_per_w = total // _NW
    ch = 128
    n_ch = b_per_w // ch
    mesh = plsc.VectorSubcoreMesh(core_axis_name="c", subcore_axis_name="s")
    out_sds = jax.ShapeDtypeStruct((total, D_MODEL), jnp.float32)

    @functools.partial(
        pl.kernel,
        mesh=mesh,
        out_type=[out_sds, out_sds],
        scratch_types=[
            pltpu.VMEM((b_per_w,), jnp.int32),
            pltpu.VMEM((ch, D_MODEL), jnp.float32),
            pltpu.SemaphoreType.DMA,
        ],
    )
    def gather_kernel(kf_hbm, vf_hbm, idx_hbm, outk_hbm, outv_hbm,
                      idx_v, buf, sem):
        wid = lax.axis_index("s") * _NC + lax.axis_index("c")
        base = wid * b_per_w
        pltpu.sync_copy(idx_hbm.at[pl.ds(base, b_per_w)], idx_v)

        def body(c, carry):
            off = base + c * ch
            isl = idx_v.at[pl.ds(c * ch, ch)]
            pltpu.async_copy(kf_hbm.at[isl], buf, sem).wait()
            pltpu.sync_copy(buf, outk_hbm.at[pl.ds(off, ch)])
            pltpu.async_copy(vf_hbm.at[isl], buf, sem).wait()
            pltpu.sync_copy(buf, outv_hbm.at[pl.ds(off, ch)])
            return carry

        lax.fori_loop(0, n_ch, body, 0)

    return gather_kernel(kf, vf, gidx)


def _attn_body(q_ref, kn_ref, vn_ref, m1_ref, m2_ref, wo_ref, bo_ref, out_ref,
               *, rows):
    scale = jnp.float32(HEAD_DIM ** -0.5)
    q3 = jnp.broadcast_to(q_ref[...][:, None, :], (rows, K_NB, D_MODEL))
    p = (q3 * kn_ref[...]).reshape(rows * K_NB, D_MODEL)
    dot = lambda a, b: lax.dot_general(
        a, b, (((1,), (0,)), ((), ())), preferred_element_type=jnp.float32)
    s = (dot(p, m1_ref[...]) * scale).reshape(rows, K_NB, 128)
    mx = s[:, 0, :]
    for j in range(1, K_NB):
        mx = jnp.maximum(mx, s[:, j, :])
    e = jnp.exp(s - mx[:, None, :])
    den = e[:, 0, :]
    for j in range(1, K_NB):
        den = den + e[:, j, :]
    w = (e / den[:, None, :]).reshape(rows * K_NB, 128)
    wexp = dot(w, m2_ref[...])
    o = (wexp * vn_ref[...].reshape(rows * K_NB, D_MODEL)).reshape(
        rows, K_NB, D_MODEL)
    acc = o[:, 0, :]
    for j in range(1, K_NB):
        acc = acc + o[:, j, :]
    out_ref[...] = dot(acc, wo_ref[...]) + bo_ref[...]


def _attention(q, kn, vn, Wo, bo):
    m = q.shape[0]
    rows = 128
    grid = (m // rows,)
    kn3 = kn.reshape(m, K_NB, D_MODEL)
    vn3 = vn.reshape(m, K_NB, D_MODEL)
    m1 = np.zeros((D_MODEL, 128), np.float32)
    for h in range(N_HEADS):
        m1[h * HEAD_DIM:(h + 1) * HEAD_DIM, h] = 1.0
    m2 = np.zeros((128, D_MODEL), np.float32)
    for h in range(N_HEADS):
        m2[h, h * HEAD_DIM:(h + 1) * HEAD_DIM] = 1.0
    return pl.pallas_call(
        functools.partial(_attn_body, rows=rows),
        grid=grid,
        in_specs=[
            pl.BlockSpec((rows, D_MODEL), lambda i: (i, 0)),
            pl.BlockSpec((rows, K_NB, D_MODEL), lambda i: (i, 0, 0)),
            pl.BlockSpec((rows, K_NB, D_MODEL), lambda i: (i, 0, 0)),
            pl.BlockSpec((D_MODEL, 128), lambda i: (0, 0)),
            pl.BlockSpec((128, D_MODEL), lambda i: (0, 0)),
            pl.BlockSpec((D_MODEL, D_MODEL), lambda i: (0, 0)),
            pl.BlockSpec((1, D_MODEL), lambda i: (0, 0)),
        ],
        out_specs=pl.BlockSpec((rows, D_MODEL), lambda i: (i, 0)),
        out_shape=jax.ShapeDtypeStruct((m, D_MODEL), jnp.float32),
    )(q, kn3, vn3, jnp.asarray(m1), jnp.asarray(m2), Wo, bo.reshape(1, -1))


def kernel(x, coordinate, Wq, bq, Wk, bk, Wv, bv, Wo, bo):
    B, N, C = x.shape
    xf = x.reshape(B * N, C)
    q, kf, vf = _qkv(xf, Wq, bq, Wk, bk, Wv, bv)
    knn_idx = _knn(coordinate)
    kn, vn = _gather_sc(kf, vf, knn_idx.reshape(-1))
    out = _attention(q, kn, vn, Wo, bo)
    return out.reshape(B, N, C)


# speedup vs baseline: 12.5818x; 1.1752x over previous
import functools

import jax
import jax.numpy as jnp
import numpy as np
from jax import lax
from jax.experimental import pallas as pl
from jax.experimental.pallas import tpu as pltpu
from jax.experimental.pallas import tpu_sc as plsc

D_MODEL = 512
N_HEADS = 8
HEAD_DIM = D_MODEL // N_HEADS
K_NB = 16
NEG_BIG = 3.0e38
IDX_BIG = 2 ** 30


def _pack_bf16_pair(y):
    h = y.shape[1] // 2
    lo = lax.bitcast_convert_type(y[:, :h].astype(jnp.bfloat16).astype(
        jnp.float32), jnp.int32)
    hi = lax.bitcast_convert_type(y[:, h:].astype(jnp.bfloat16).astype(
        jnp.float32), jnp.int32)
    return lax.shift_right_logical(lo, 16) | (hi & jnp.int32(-65536))


def _unpack_bf16_pair(w):
    lo = lax.bitcast_convert_type(lax.shift_left(w, 16), jnp.float32)
    hi = lax.bitcast_convert_type(w & jnp.int32(-65536), jnp.float32)
    return jnp.concatenate([lo, hi], axis=-1)


def _qkv_body(x_ref, wq_ref, bq_ref, wk_ref, bk_ref, wv_ref, bv_ref,
              q_ref, k_ref, v_ref):
    x = x_ref[...]
    dot = lambda a, b: lax.dot_general(
        a, b, (((1,), (0,)), ((), ())), preferred_element_type=jnp.float32)
    q_ref[...] = dot(x, wq_ref[...]) + bq_ref[...]
    k_ref[...] = _pack_bf16_pair(dot(x, wk_ref[...]) + bk_ref[...])
    v_ref[...] = _pack_bf16_pair(dot(x, wv_ref[...]) + bv_ref[...])


def _qkv(xf, Wq, bq, Wk, bk, Wv, bv):
    m = xf.shape[0]
    blk = 512
    grid = (m // blk,)
    row_spec = pl.BlockSpec((blk, D_MODEL), lambda i: (i, 0))
    pk_spec = pl.BlockSpec((blk, D_MODEL // 2), lambda i: (i, 0))
    full_spec = pl.BlockSpec((D_MODEL, D_MODEL), lambda i: (0, 0))
    bias_spec = pl.BlockSpec((1, D_MODEL), lambda i: (0, 0))
    out_f = jax.ShapeDtypeStruct((m, D_MODEL), jnp.float32)
    out_p = jax.ShapeDtypeStruct((m, D_MODEL // 2), jnp.int32)
    return pl.pallas_call(
        _qkv_body,
        grid=grid,
        in_specs=[row_spec, full_spec, bias_spec, full_spec, bias_spec,
                  full_spec, bias_spec],
        out_specs=[row_spec, pk_spec, pk_spec],
        out_shape=[out_f, out_p, out_p],
    )(xf, Wq, bq.reshape(1, -1), Wk, bk.reshape(1, -1), Wv, bv.reshape(1, -1))


def _knn_body(cp_ref, cpt_ref, idx_ref, *, rows, n):
    b = pl.program_id(0)
    cb = cp_ref[0]
    cft = cpt_ref[0]
    g = lax.dot_general(cb, cft, (((1,), (0,)), ((), ())),
                        preferred_element_type=jnp.float32)
    sqb = jnp.sum(cb * cb, axis=1, keepdims=True)
    sqf = jnp.sum(cft * cft, axis=0, keepdims=True)
    d2 = sqb + sqf - 2.0 * g
    iota = lax.broadcasted_iota(jnp.int32, (rows, n), 1)
    cols = []
    for _ in range(K_NB):
        m = jnp.min(d2, axis=1, keepdims=True)
        cand = jnp.where(d2 == m, iota, IDX_BIG)
        amin = jnp.min(cand, axis=1, keepdims=True)
        cols.append(amin)
        d2 = jnp.where(iota == amin, NEG_BIG, d2)
    idx_ref[...] = jnp.concatenate(cols, axis=1) + b * n


def _knn(coordinate):
    B, n, _ = coordinate.shape
    rows = 256
    cpad = jnp.zeros((B, n, 128), jnp.float32).at[:, :, :3].set(coordinate)
    cpadT = cpad.transpose(0, 2, 1)
    grid = (B, n // rows)
    return pl.pallas_call(
        functools.partial(_knn_body, rows=rows, n=n),
        grid=grid,
        in_specs=[
            pl.BlockSpec((1, rows, 128), lambda b, i: (b, i, 0)),
            pl.BlockSpec((1, 128, n), lambda b, i: (b, 0, 0)),
        ],
        out_specs=pl.BlockSpec((rows, K_NB), lambda b, i: (b * (n // rows) + i, 0)),
        out_shape=jax.ShapeDtypeStruct((B * n, K_NB), jnp.int32),
    )(cpad, cpadT)


_NC = 2
_NS = 16
_NW = _NC * _NS


def _gather_sc(kb, vb, gidx):
    total = gidx.shape[0]
    d = kb.shape[1]
    b_per_w = total // _NW
    ch = 128
    n_ch = b_per_w // ch
    mesh = plsc.VectorSubcoreMesh(core_axis_name="c", subcore_axis_name="s")
    out_sds = jax.ShapeDtypeStruct((total, d), jnp.int32)

    @functools.partial(
        pl.kernel,
        mesh=mesh,
        out_type=[out_sds, out_sds],
        scratch_types=[
            pltpu.VMEM((b_per_w,), jnp.int32),
            pltpu.VMEM((ch, d), jnp.int32),
            pltpu.VMEM((ch, d), jnp.int32),
            pltpu.SemaphoreType.DMA,
            pltpu.SemaphoreType.DMA,
        ],
    )
    def gather_kernel(kb_hbm, vb_hbm, idx_hbm, outk_hbm, outv_hbm,
                      idx_v, bk, bv, sem_g, sem_s):
        wid = lax.axis_index("s") * _NC + lax.axis_index("c")
        base = wid * b_per_w
        pltpu.sync_copy(idx_hbm.at[pl.ds(base, b_per_w)], idx_v)

        def body(c, carry):
            off = base + c * ch
            isl = idx_v.at[pl.ds(c * ch, ch)]
            gk = pltpu.async_copy(kb_hbm.at[isl], bk, sem_g)
            gv = pltpu.async_copy(vb_hbm.at[isl], bv, sem_g)
            gk.wait()
            gv.wait()
            sk = pltpu.async_copy(bk, outk_hbm.at[pl.ds(off, ch)], sem_s)
            sv = pltpu.async_copy(bv, outv_hbm.at[pl.ds(off, ch)], sem_s)
            sk.wait()
            sv.wait()
            return carry

        lax.fori_loop(0, n_ch, body, 0)

    return gather_kernel(kb, vb, gidx)


def _attn_body(q_ref, kn_ref, vn_ref, m1_ref, m2_ref, wo_ref, bo_ref, out_ref,
               *, rows):
    scale = jnp.float32(HEAD_DIM ** -0.5)
    q3 = jnp.broadcast_to(q_ref[...][:, None, :], (rows, K_NB, D_MODEL))
    kn = _unpack_bf16_pair(kn_ref[...])
    p = (q3 * kn).reshape(rows * K_NB, D_MODEL)
    dot = lambda a, b: lax.dot_general(
        a, b, (((1,), (0,)), ((), ())), preferred_element_type=jnp.float32)
    s = (dot(p, m1_ref[...]) * scale).reshape(rows, K_NB, 128)
    mx = s[:, 0, :]
    for j in range(1, K_NB):
        mx = jnp.maximum(mx, s[:, j, :])
    e = jnp.exp(s - mx[:, None, :])
    den = e[:, 0, :]
    for j in range(1, K_NB):
        den = den + e[:, j, :]
    w = (e / den[:, None, :]).reshape(rows * K_NB, 128)
    wexp = dot(w, m2_ref[...])
    vn = _unpack_bf16_pair(vn_ref[...])
    o = (wexp * vn.reshape(rows * K_NB, D_MODEL)).reshape(
        rows, K_NB, D_MODEL)
    acc = o[:, 0, :]
    for j in range(1, K_NB):
        acc = acc + o[:, j, :]
    out_ref[...] = dot(acc, wo_ref[...]) + bo_ref[...]


def _attention(q, kn, vn, Wo, bo):
    m = q.shape[0]
    rows = 128
    grid = (m // rows,)
    kn3 = kn.reshape(m, K_NB, D_MODEL // 2)
    vn3 = vn.reshape(m, K_NB, D_MODEL // 2)
    m1 = np.zeros((D_MODEL, 128), np.float32)
    for h in range(N_HEADS):
        m1[h * HEAD_DIM:(h + 1) * HEAD_DIM, h] = 1.0
    m2 = np.zeros((128, D_MODEL), np.float32)
    for h in range(N_HEADS):
        m2[h, h * HEAD_DIM:(h + 1) * HEAD_DIM] = 1.0
    return pl.pallas_call(
        functools.partial(_attn_body, rows=rows),
        grid=grid,
        in_specs=[
            pl.BlockSpec((rows, D_MODEL), lambda i: (i, 0)),
            pl.BlockSpec((rows, K_NB, D_MODEL // 2), lambda i: (i, 0, 0)),
            pl.BlockSpec((rows, K_NB, D_MODEL // 2), lambda i: (i, 0, 0)),
            pl.BlockSpec((D_MODEL, 128), lambda i: (0, 0)),
            pl.BlockSpec((128, D_MODEL), lambda i: (0, 0)),
            pl.BlockSpec((D_MODEL, D_MODEL), lambda i: (0, 0)),
            pl.BlockSpec((1, D_MODEL), lambda i: (0, 0)),
        ],
        out_specs=pl.BlockSpec((rows, D_MODEL), lambda i: (i, 0)),
        out_shape=jax.ShapeDtypeStruct((m, D_MODEL), jnp.float32),
    )(q, kn3, vn3, jnp.asarray(m1), jnp.asarray(m2), Wo, bo.reshape(1, -1))


def kernel(x, coordinate, Wq, bq, Wk, bk, Wv, bv, Wo, bo):
    B, N, C = x.shape
    xf = x.reshape(B * N, C)
    q, kb, vb = _qkv(xf, Wq, bq, Wk, bk, Wv, bv)
    knn_idx = _knn(coordinate)
    kn, vn = _gather_sc(kb, vb, knn_idx.reshape(-1))
    out = _attention(q, kn, vn, Wo, bo)
    return out.reshape(B, N, C)


# speedup vs baseline: 12.6792x; 1.0077x over previous
import functools

import jax
import jax.numpy as jnp
import numpy as np
from jax import lax
from jax.experimental import pallas as pl
from jax.experimental.pallas import tpu as pltpu
from jax.experimental.pallas import tpu_sc as plsc

D_MODEL = 512
N_HEADS = 8
HEAD_DIM = D_MODEL // N_HEADS
K_NB = 16
NEG_BIG = 3.0e38
IDX_BIG = 2 ** 30


def _pack_bf16_pair(y):
    h = y.shape[1] // 2
    lo = lax.bitcast_convert_type(y[:, :h].astype(jnp.bfloat16).astype(
        jnp.float32), jnp.int32)
    hi = lax.bitcast_convert_type(y[:, h:].astype(jnp.bfloat16).astype(
        jnp.float32), jnp.int32)
    return lax.shift_right_logical(lo, 16) | (hi & jnp.int32(-65536))


def _unpack_bf16_pair(w):
    lo = lax.bitcast_convert_type(lax.shift_left(w, 16), jnp.float32)
    hi = lax.bitcast_convert_type(w & jnp.int32(-65536), jnp.float32)
    return jnp.concatenate([lo, hi], axis=-1)


def _qkv_body(x_ref, wq_ref, bq_ref, wk_ref, bk_ref, wv_ref, bv_ref,
              q_ref, kv_ref):
    x = x_ref[...]
    dot = lambda a, b: lax.dot_general(
        a, b, (((1,), (0,)), ((), ())), preferred_element_type=jnp.float32)
    q_ref[...] = dot(x, wq_ref[...]) + bq_ref[...]
    kp = _pack_bf16_pair(dot(x, wk_ref[...]) + bk_ref[...])
    vp = _pack_bf16_pair(dot(x, wv_ref[...]) + bv_ref[...])
    kv_ref[...] = jnp.concatenate([kp, vp], axis=1)


def _qkv(xf, Wq, bq, Wk, bk, Wv, bv):
    m = xf.shape[0]
    blk = 512
    grid = (m // blk,)
    row_spec = pl.BlockSpec((blk, D_MODEL), lambda i: (i, 0))
    full_spec = pl.BlockSpec((D_MODEL, D_MODEL), lambda i: (0, 0))
    bias_spec = pl.BlockSpec((1, D_MODEL), lambda i: (0, 0))
    out_f = jax.ShapeDtypeStruct((m, D_MODEL), jnp.float32)
    out_p = jax.ShapeDtypeStruct((m, D_MODEL), jnp.int32)
    return pl.pallas_call(
        _qkv_body,
        grid=grid,
        in_specs=[row_spec, full_spec, bias_spec, full_spec, bias_spec,
                  full_spec, bias_spec],
        out_specs=[row_spec, row_spec],
        out_shape=[out_f, out_p],
    )(xf, Wq, bq.reshape(1, -1), Wk, bk.reshape(1, -1), Wv, bv.reshape(1, -1))


def _knn_body(cp_ref, cpt_ref, idx_ref, *, rows, n):
    b = pl.program_id(0)
    cb = cp_ref[0]
    cft = cpt_ref[0]
    g = lax.dot_general(cb, cft, (((1,), (0,)), ((), ())),
                        preferred_element_type=jnp.float32)
    sqb = jnp.sum(cb * cb, axis=1, keepdims=True)
    sqf = jnp.sum(cft * cft, axis=0, keepdims=True)
    d2 = sqb + sqf - 2.0 * g
    iota = lax.broadcasted_iota(jnp.int32, (rows, n), 1)
    cols = []
    for _ in range(K_NB):
        m = jnp.min(d2, axis=1, keepdims=True)
        cand = jnp.where(d2 == m, iota, IDX_BIG)
        amin = jnp.min(cand, axis=1, keepdims=True)
        cols.append(amin)
        d2 = jnp.where(iota == amin, NEG_BIG, d2)
    idx_ref[...] = jnp.concatenate(cols, axis=1) + b * n


def _knn(coordinate):
    B, n, _ = coordinate.shape
    rows = 256
    cpad = jnp.zeros((B, n, 128), jnp.float32).at[:, :, :3].set(coordinate)
    cpadT = cpad.transpose(0, 2, 1)
    grid = (B, n // rows)
    return pl.pallas_call(
        functools.partial(_knn_body, rows=rows, n=n),
        grid=grid,
        in_specs=[
            pl.BlockSpec((1, rows, 128), lambda b, i: (b, i, 0)),
            pl.BlockSpec((1, 128, n), lambda b, i: (b, 0, 0)),
        ],
        out_specs=pl.BlockSpec((rows, K_NB), lambda b, i: (b * (n // rows) + i, 0)),
        out_shape=jax.ShapeDtypeStruct((B * n, K_NB), jnp.int32),
    )(cpad, cpadT)


_NC = 2
_NS = 16
_NW = _NC * _NS


def _gather_sc(kv, gidx):
    total = gidx.shape[0]
    d = kv.shape[1]
    b_per_w = total // _NW
    ch = 64
    n_pair = b_per_w // (2 * ch)
    mesh = plsc.VectorSubcoreMesh(core_axis_name="c", subcore_axis_name="s")

    @functools.partial(
        pl.kernel,
        mesh=mesh,
        out_type=jax.ShapeDtypeStruct((total, d), jnp.int32),
        scratch_types=[
            pltpu.VMEM((b_per_w,), jnp.int32),
            pltpu.VMEM((ch, d), jnp.int32),
            pltpu.VMEM((ch, d), jnp.int32),
            pltpu.SemaphoreType.DMA,
            pltpu.SemaphoreType.DMA,
            pltpu.SemaphoreType.DMA,
            pltpu.SemaphoreType.DMA,
        ],
    )
    def gather_kernel(kv_hbm, idx_hbm, out_hbm, idx_v, b0, b1,
                      sg0, sg1, ss0, ss1):
        wid = lax.axis_index("s") * _NC + lax.axis_index("c")
        base = wid * b_per_w
        pltpu.sync_copy(idx_hbm.at[pl.ds(base, b_per_w)], idx_v)

        def gath(c, buf, sem):
            pltpu.async_copy(kv_hbm.at[idx_v.at[pl.ds(c * ch, ch)]], buf, sem)

        def scat(c, buf, sem):
            pltpu.async_copy(buf, out_hbm.at[pl.ds(base + c * ch, ch)], sem)

        def drain(sem, buf):
            pltpu.make_async_copy(kv_hbm.at[pl.ds(0, ch)], buf, sem).wait()

        gath(0, b0, sg0)

        def body(i, carry):
            c0 = 2 * i
            c1 = c0 + 1
            drain(sg0, b0)

            @pl.when(i > 0)
            def _():
                drain(ss1, b1)

            gath(c1, b1, sg1)
            scat(c0, b0, ss0)
            drain(sg1, b1)
            drain(ss0, b0)

            @pl.when(i < n_pair - 1)
            def _():
                gath(c0 + 2, b0, sg0)

            scat(c1, b1, ss1)
            return carry

        lax.fori_loop(0, n_pair, body, 0)
        drain(ss1, b1)

    return gather_kernel(kv, gidx)


def _attn_body(q_ref, kv_ref, m1_ref, m2_ref, wo_ref, bo_ref, out_ref,
               *, rows):
    scale = jnp.float32(HEAD_DIM ** -0.5)
    kvw = kv_ref[...]
    kn = _unpack_bf16_pair(kvw[:, :, :D_MODEL // 2])
    vn = _unpack_bf16_pair(kvw[:, :, D_MODEL // 2:])
    q3 = jnp.broadcast_to(q_ref[...][:, None, :], (rows, K_NB, D_MODEL))
    p = (q3 * kn).reshape(rows * K_NB, D_MODEL)
    dot = lambda a, b: lax.dot_general(
        a, b, (((1,), (0,)), ((), ())), preferred_element_type=jnp.float32)
    s = (dot(p.astype(jnp.bfloat16), m1_ref[...]) * scale).reshape(
        rows, K_NB, 128)
    mx = s[:, 0, :]
    for j in range(1, K_NB):
        mx = jnp.maximum(mx, s[:, j, :])
    e = jnp.exp(s - mx[:, None, :])
    den = e[:, 0, :]
    for j in range(1, K_NB):
        den = den + e[:, j, :]
    w = (e / den[:, None, :]).reshape(rows * K_NB, 128)
    wexp = dot(w.astype(jnp.bfloat16), m2_ref[...])
    o = (wexp * vn.reshape(rows * K_NB, D_MODEL)).reshape(
        rows, K_NB, D_MODEL)
    acc = o[:, 0, :]
    for j in range(1, K_NB):
        acc = acc + o[:, j, :]
    out_ref[...] = dot(acc, wo_ref[...]) + bo_ref[...]


def _attention(q, kvn, Wo, bo):
    m = q.shape[0]
    rows = 128
    grid = (m // rows,)
    kv3 = kvn.reshape(m, K_NB, D_MODEL)
    m1 = np.zeros((D_MODEL, 128), np.float32)
    for h in range(N_HEADS):
        m1[h * HEAD_DIM:(h + 1) * HEAD_DIM, h] = 1.0
    m2 = np.zeros((128, D_MODEL), np.float32)
    for h in range(N_HEADS):
        m2[h, h * HEAD_DIM:(h + 1) * HEAD_DIM] = 1.0
    return pl.pallas_call(
        functools.partial(_attn_body, rows=rows),
        grid=grid,
        in_specs=[
            pl.BlockSpec((rows, D_MODEL), lambda i: (i, 0)),
            pl.BlockSpec((rows, K_NB, D_MODEL), lambda i: (i, 0, 0)),
            pl.BlockSpec((D_MODEL, 128), lambda i: (0, 0)),
            pl.BlockSpec((128, D_MODEL), lambda i: (0, 0)),
            pl.BlockSpec((D_MODEL, D_MODEL), lambda i: (0, 0)),
            pl.BlockSpec((1, D_MODEL), lambda i: (0, 0)),
        ],
        out_specs=pl.BlockSpec((rows, D_MODEL), lambda i: (i, 0)),
        out_shape=jax.ShapeDtypeStruct((m, D_MODEL), jnp.float32),
    )(q, kv3, jnp.asarray(m1, jnp.bfloat16), jnp.asarray(m2, jnp.bfloat16),
      Wo, bo.reshape(1, -1))


def kernel(x, coordinate, Wq, bq, Wk, bk, Wv, bv, Wo, bo):
    B, N, C = x.shape
    xf = x.reshape(B * N, C)
    q, kv = _qkv(xf, Wq, bq, Wk, bk, Wv, bv)
    knn_idx = _knn(coordinate)
    kvn = _gather_sc(kv, knn_idx.reshape(-1))
    out = _attention(q, kvn, Wo, bo)
    return out.reshape(B, N, C)


# speedup vs baseline: 14.9866x; 1.1820x over previous
import functools

import jax
import jax.numpy as jnp
import numpy as np
from jax import lax
from jax.experimental import pallas as pl
from jax.experimental.pallas import tpu as pltpu
from jax.experimental.pallas import tpu_sc as plsc

D_MODEL = 512
N_HEADS = 8
HEAD_DIM = D_MODEL // N_HEADS
K_NB = 16
NEG_BIG = 3.0e38
IDX_BIG = 2 ** 30


def _pack_bf16_pair(y):
    h = y.shape[1] // 2
    lo = lax.bitcast_convert_type(y[:, :h].astype(jnp.bfloat16).astype(
        jnp.float32), jnp.int32)
    hi = lax.bitcast_convert_type(y[:, h:].astype(jnp.bfloat16).astype(
        jnp.float32), jnp.int32)
    return lax.shift_right_logical(lo, 16) | (hi & jnp.int32(-65536))


def _unpack_bf16_pair(w):
    lo = lax.bitcast_convert_type(lax.shift_left(w, 16), jnp.float32)
    hi = lax.bitcast_convert_type(w & jnp.int32(-65536), jnp.float32)
    return jnp.concatenate([lo, hi], axis=-1)


def _qkv_body(x_ref, wq_ref, bq_ref, wk_ref, bk_ref, wv_ref, bv_ref,
              q_ref, kv_ref):
    x = x_ref[...]
    dot = lambda a, b: lax.dot_general(
        a, b, (((1,), (0,)), ((), ())), preferred_element_type=jnp.float32)
    q_ref[...] = dot(x, wq_ref[...]) + bq_ref[...]
    kp = _pack_bf16_pair(dot(x, wk_ref[...]) + bk_ref[...])
    vp = _pack_bf16_pair(dot(x, wv_ref[...]) + bv_ref[...])
    kv_ref[...] = jnp.concatenate([kp, vp], axis=1)


def _qkv(xf, Wq, bq, Wk, bk, Wv, bv):
    m = xf.shape[0]
    blk = 512
    grid = (m // blk,)
    row_spec = pl.BlockSpec((blk, D_MODEL), lambda i: (i, 0))
    full_spec = pl.BlockSpec((D_MODEL, D_MODEL), lambda i: (0, 0))
    bias_spec = pl.BlockSpec((1, D_MODEL), lambda i: (0, 0))
    out_f = jax.ShapeDtypeStruct((m, D_MODEL), jnp.float32)
    out_p = jax.ShapeDtypeStruct((m, D_MODEL), jnp.int32)
    return pl.pallas_call(
        _qkv_body,
        grid=grid,
        in_specs=[row_spec, full_spec, bias_spec, full_spec, bias_spec,
                  full_spec, bias_spec],
        out_specs=[row_spec, row_spec],
        out_shape=[out_f, out_p],
    )(xf, Wq, bq.reshape(1, -1), Wk, bk.reshape(1, -1), Wv, bv.reshape(1, -1))


def _knn_body(cp_ref, cpt_ref, idx_ref, *, rows, n):
    b = pl.program_id(0)
    cb = cp_ref[0]
    cft = cpt_ref[0]
    g = lax.dot_general(cb, cft, (((1,), (0,)), ((), ())),
                        preferred_element_type=jnp.float32)
    sqb = jnp.sum(cb * cb, axis=1, keepdims=True)
    sqf = jnp.sum(cft * cft, axis=0, keepdims=True)
    d2 = sqb + sqf - 2.0 * g
    bits = lax.bitcast_convert_type(jnp.maximum(d2, 0.0), jnp.int32)
    iota = lax.broadcasted_iota(jnp.int32, (rows, n), 1)
    pk = (bits & jnp.int32(-4096)) | iota
    cols = []
    for _ in range(K_NB - 1):
        m = jnp.min(pk, axis=1, keepdims=True)
        cols.append(m & jnp.int32(4095))
        pk = jnp.where(pk == m, IDX_BIG, pk)
    d2m = jnp.where(pk == IDX_BIG, NEG_BIG, d2)
    m16 = jnp.min(d2m, axis=1, keepdims=True)
    cand = jnp.where(d2m == m16, iota, IDX_BIG)
    cols.append(jnp.min(cand, axis=1, keepdims=True))
    idx_ref[...] = jnp.concatenate(cols, axis=1) + b * n


def _knn(coordinate):
    B, n, _ = coordinate.shape
    rows = 256
    cpad = jnp.zeros((B, n, 128), jnp.float32).at[:, :, :3].set(coordinate)
    cpadT = cpad.transpose(0, 2, 1)
    grid = (B, n // rows)
    return pl.pallas_call(
        functools.partial(_knn_body, rows=rows, n=n),
        grid=grid,
        in_specs=[
            pl.BlockSpec((1, rows, 128), lambda b, i: (b, i, 0)),
            pl.BlockSpec((1, 128, n), lambda b, i: (b, 0, 0)),
        ],
        out_specs=pl.BlockSpec((rows, K_NB), lambda b, i: (b * (n // rows) + i, 0)),
        out_shape=jax.ShapeDtypeStruct((B * n, K_NB), jnp.int32),
    )(cpad, cpadT)


_NC = 2
_NS = 16
_NW = _NC * _NS


def _gather_sc(kv, gidx):
    total = gidx.shape[0]
    d = kv.shape[1]
    b_per_w = total // _NW
    ch = 64
    n_pair = b_per_w // (2 * ch)
    mesh = plsc.VectorSubcoreMesh(core_axis_name="c", subcore_axis_name="s")

    @functools.partial(
        pl.kernel,
        mesh=mesh,
        out_type=jax.ShapeDtypeStruct((total, d), jnp.int32),
        scratch_types=[
            pltpu.VMEM((b_per_w,), jnp.int32),
            pltpu.VMEM((ch, d), jnp.int32),
            pltpu.VMEM((ch, d), jnp.int32),
            pltpu.SemaphoreType.DMA,
            pltpu.SemaphoreType.DMA,
            pltpu.SemaphoreType.DMA,
            pltpu.SemaphoreType.DMA,
        ],
    )
    def gather_kernel(kv_hbm, idx_hbm, out_hbm, idx_v, b0, b1,
                      sg0, sg1, ss0, ss1):
        wid = lax.axis_index("s") * _NC + lax.axis_index("c")
        base = wid * b_per_w
        pltpu.sync_copy(idx_hbm.at[pl.ds(base, b_per_w)], idx_v)

        def gath(c, buf, sem):
            pltpu.async_copy(kv_hbm.at[idx_v.at[pl.ds(c * ch, ch)]], buf, sem)

        def scat(c, buf, sem):
            pltpu.async_copy(buf, out_hbm.at[pl.ds(base + c * ch, ch)], sem)

        def drain(sem, buf):
            pltpu.make_async_copy(kv_hbm.at[pl.ds(0, ch)], buf, sem).wait()

        gath(0, b0, sg0)

        def body(i, carry):
            c0 = 2 * i
            c1 = c0 + 1
            drain(sg0, b0)

            @pl.when(i > 0)
            def _():
                drain(ss1, b1)

            gath(c1, b1, sg1)
            scat(c0, b0, ss0)
            drain(sg1, b1)
            drain(ss0, b0)

            @pl.when(i < n_pair - 1)
            def _():
                gath(c0 + 2, b0, sg0)

            scat(c1, b1, ss1)
            return carry

        lax.fori_loop(0, n_pair, body, 0)
        drain(ss1, b1)

    return gather_kernel(kv, gidx)


def _attn_body(q_ref, kv_ref, m1_ref, m2_ref, wo_ref, bo_ref, out_ref,
               *, rows):
    scale = jnp.float32(HEAD_DIM ** -0.5)
    kvw = kv_ref[...]
    kn = _unpack_bf16_pair(kvw[:, :, :D_MODEL // 2])
    vn = _unpack_bf16_pair(kvw[:, :, D_MODEL // 2:])
    q3 = jnp.broadcast_to(q_ref[...][:, None, :], (rows, K_NB, D_MODEL))
    p = (q3 * kn).reshape(rows * K_NB, D_MODEL)
    dot = lambda a, b: lax.dot_general(
        a, b, (((1,), (0,)), ((), ())), preferred_element_type=jnp.float32)
    s = (dot(p.astype(jnp.bfloat16), m1_ref[...]) * scale).reshape(
        rows, K_NB, 128)
    mx = s[:, 0, :]
    for j in range(1, K_NB):
        mx = jnp.maximum(mx, s[:, j, :])
    e = jnp.exp(s - mx[:, None, :])
    den = e[:, 0, :]
    for j in range(1, K_NB):
        den = den + e[:, j, :]
    w = (e / den[:, None, :]).reshape(rows * K_NB, 128)
    wexp = dot(w.astype(jnp.bfloat16), m2_ref[...])
    o = (wexp * vn.reshape(rows * K_NB, D_MODEL)).reshape(
        rows, K_NB, D_MODEL)
    acc = o[:, 0, :]
    for j in range(1, K_NB):
        acc = acc + o[:, j, :]
    out_ref[...] = dot(acc, wo_ref[...]) + bo_ref[...]


def _attention(q, kvn, Wo, bo):
    m = q.shape[0]
    rows = 128
    grid = (m // rows,)
    kv3 = kvn.reshape(m, K_NB, D_MODEL)
    m1 = np.zeros((D_MODEL, 128), np.float32)
    for h in range(N_HEADS):
        m1[h * HEAD_DIM:(h + 1) * HEAD_DIM, h] = 1.0
    m2 = np.zeros((128, D_MODEL), np.float32)
    for h in range(N_HEADS):
        m2[h, h * HEAD_DIM:(h + 1) * HEAD_DIM] = 1.0
    return pl.pallas_call(
        functools.partial(_attn_body, rows=rows),
        grid=grid,
        in_specs=[
            pl.BlockSpec((rows, D_MODEL), lambda i: (i, 0)),
            pl.BlockSpec((rows, K_NB, D_MODEL), lambda i: (i, 0, 0)),
            pl.BlockSpec((D_MODEL, 128), lambda i: (0, 0)),
            pl.BlockSpec((128, D_MODEL), lambda i: (0, 0)),
            pl.BlockSpec((D_MODEL, D_MODEL), lambda i: (0, 0)),
            pl.BlockSpec((1, D_MODEL), lambda i: (0, 0)),
        ],
        out_specs=pl.BlockSpec((rows, D_MODEL), lambda i: (i, 0)),
        out_shape=jax.ShapeDtypeStruct((m, D_MODEL), jnp.float32),
    )(q, kv3, jnp.asarray(m1, jnp.bfloat16), jnp.asarray(m2, jnp.bfloat16),
      Wo, bo.reshape(1, -1))


def kernel(x, coordinate, Wq, bq, Wk, bk, Wv, bv, Wo, bo):
    B, N, C = x.shape
    xf = x.reshape(B * N, C)
    q, kv = _qkv(xf, Wq, bq, Wk, bk, Wv, bv)
    knn_idx = _knn(coordinate)
    kvn = _gather_sc(kv, knn_idx.reshape(-1))
    out = _attention(q, kvn, Wo, bo)
    return out.reshape(B, N, C)
